# baseline (speedup 1.0000x reference)
import functools

import jax
import jax.numpy as jnp
from jax import lax
from jax.experimental import pallas as pl
from jax.experimental.pallas import tpu as pltpu
from jax.experimental.pallas import tpu_sc as plsc

_NUM_CORES = 2
_NUM_SUBCORES = 16
_NUM_WORKERS = _NUM_CORES * _NUM_SUBCORES

_CHUNK = 128


def _make_gather(vocab: int, batch: int, dim: int):
  b_per_w = batch // _NUM_WORKERS
  n_chunks = b_per_w // _CHUNK
  mesh = plsc.VectorSubcoreMesh(core_axis_name="c", subcore_axis_name="s")

  @functools.partial(
      pl.kernel,
      mesh=mesh,
      out_type=jax.ShapeDtypeStruct((batch, dim), jnp.float32),
      compiler_params=pltpu.CompilerParams(use_tc_tiling_on_sc=False),
      scratch_types=[
          pltpu.VMEM((n_chunks, _CHUNK), jnp.int32),
          pltpu.VMEM((n_chunks, _CHUNK, dim), jnp.float32),
          pltpu.SemaphoreType.DMA,
      ],
  )
  def gather(idx_hbm, table_hbm, out_hbm, idx_v, rows_v, sem):
    wid = lax.axis_index("s") * _NUM_CORES + lax.axis_index("c")
    base = wid * b_per_w
    pltpu.sync_copy(idx_hbm.at[wid], idx_v)
    copies = []
    for j in range(n_chunks):
      copies.append(
          pltpu.async_copy(table_hbm.at[idx_v.at[j]], rows_v.at[j], sem))
    for j in range(n_chunks):
      copies[j].wait()
    for j in range(n_chunks):
      pltpu.sync_copy(rows_v.at[j],
                      out_hbm.at[pl.ds(base + j * _CHUNK, _CHUNK)])

  return gather


def kernel(idx, text_embeds):
  vocab, dim = text_embeds.shape
  (batch,) = idx.shape
  idx_r = idx.astype(jnp.int32).reshape(
      _NUM_WORKERS, batch // (_NUM_WORKERS * _CHUNK), _CHUNK)
  return _make_gather(vocab, batch, dim)(idx_r, text_embeds)


# speedup vs baseline: 1.0012x; 1.0012x over previous
import functools

import jax
import jax.numpy as jnp
from jax import lax
from jax.experimental import pallas as pl
from jax.experimental.pallas import tpu as pltpu
from jax.experimental.pallas import tpu_sc as plsc

_NUM_CORES = 2
_NUM_SUBCORES = 16
_NUM_WORKERS = _NUM_CORES * _NUM_SUBCORES

_CHUNK = 512


def _make_gather(vocab: int, batch: int, dim: int):
  b_per_w = batch // _NUM_WORKERS
  n_chunks = b_per_w // _CHUNK
  mesh = plsc.VectorSubcoreMesh(core_axis_name="c", subcore_axis_name="s")

  @functools.partial(
      pl.kernel,
      mesh=mesh,
      out_type=jax.ShapeDtypeStruct((batch, dim), jnp.float32),
      compiler_params=pltpu.CompilerParams(use_tc_tiling_on_sc=False),
      scratch_types=[
          pltpu.VMEM((n_chunks, _CHUNK), jnp.int32),
          pltpu.VMEM((n_chunks, _CHUNK, dim), jnp.float32),
          pltpu.SemaphoreType.DMA,
      ],
  )
  def gather(idx_hbm, table_hbm, out_hbm, idx_v, rows_v, sem):
    wid = lax.axis_index("s") * _NUM_CORES + lax.axis_index("c")
    base = wid * b_per_w
    pltpu.sync_copy(idx_hbm.at[wid], idx_v)
    copies = []
    for j in range(n_chunks):
      copies.append(
          pltpu.async_copy(table_hbm.at[idx_v.at[j]], rows_v.at[j], sem))
    for j in range(n_chunks):
      copies[j].wait()
      pltpu.sync_copy(rows_v.at[j],
                      out_hbm.at[pl.ds(base + j * _CHUNK, _CHUNK)])

  return gather


def kernel(idx, text_embeds):
  vocab, dim = text_embeds.shape
  (batch,) = idx.shape
  idx_r = idx.astype(jnp.int32).reshape(
      _NUM_WORKERS, batch // (_NUM_WORKERS * _CHUNK), _CHUNK)
  return _make_gather(vocab, batch, dim)(idx_r, text_embeds)
